# baseline (speedup 1.0000x reference)
import functools

import jax
import jax.numpy as jnp
from jax import lax
from jax.experimental import pallas as pl
from jax.experimental.pallas import tpu as pltpu
from jax.experimental.pallas import tpu_sc as plsc

_B, _R, _K, _S = 16, 4, 8, 200
_D = 32
_N = _B * _R * _K * _S
_TPW = 3200
_WIN = 65536
_SW = 1024
_NSP = _WIN // _SW
_TAIL = 999936
_CLAMP = _TAIL - _SW
_CAP = _TPW


def _lsr(x, n):
    return lax.shift_right_logical(x, n)


@functools.cache
def _make_kernel():
    info = plsc.get_sparse_core_info()
    nc, ns = info.num_cores, info.num_subcores
    assert nc == 2 and ns == 16
    nw = nc * ns
    mesh = plsc.VectorSubcoreMesh(core_axis_name="c", subcore_axis_name="s")

    @functools.partial(
        pl.kernel,
        mesh=mesh,
        compiler_params=pltpu.CompilerParams(needs_layout_passes=False),
        out_type=(
            jax.ShapeDtypeStruct((_N + 16, 128), jnp.float32),
            jax.ShapeDtypeStruct((nw * 16 * _CAP,), jnp.int32),
            jax.ShapeDtypeStruct((nw * 128,), jnp.int32),
        ),
        scratch_types=[
            pltpu.VMEM((16, 384), jnp.int32),
            pltpu.VMEM((3328,), jnp.int32),
            pltpu.VMEM((3328,), jnp.int32),
            pltpu.VMEM((16, _SW), jnp.int32),
            pltpu.VMEM((4096,), jnp.int32),
            pltpu.VMEM((128,), jnp.int32),
            pltpu.VMEM((_D, _SW), jnp.float32),
            pltpu.VMEM((_D, _SW), jnp.float32),
            pltpu.VMEM((16, 128), jnp.float32),
            pltpu.VMEM((128, 128), jnp.float32),
            pltpu.VMEM((64, _D), jnp.float32),
            pltpu.SemaphoreType.DMA,
            pltpu.SemaphoreType.DMA,
        ],
    )
    def gather_kernel(
        tab_t, idxp, tail_tab, out1, bucket, counts,
        idx_v, binbuf, hitbuf, work_v, cnt_v, mycnt_v,
        strip0, strip1, rowbuf, rowring, tail_v, sem, ssem,
    ):
        c = lax.axis_index("c")
        me = lax.axis_index("s")
        g = c * ns + me
        iota = lax.iota(jnp.int32, 16)
        zeros16 = jnp.zeros((16,), jnp.int32)

        tok_base = g * _TPW
        for row in range(16):
            align = (row * 200) // 128 * 128
            pltpu.sync_copy(
                idxp.at[pl.ds(tok_base + align, 384)], idx_v.at[row]
            )
        pltpu.sync_copy(tail_tab, tail_v)

        def load_ids(row, grp):
            shift = (row * 200) % 128
            ids = plsc.load_gather(
                idx_v, [zeros16 + row, shift + grp * 16 + iota]
            )
            pos = row * 200 + grp * 16 + iota
            valid = (grp * 16 + iota) < _S
            return ids, pos, valid

        for w in range(16):

            def bin_grp(gg, cur, w=w):
                row = gg // 13
                grp = gg - row * 13
                ids, pos, valid = load_ids(row, grp)
                m = valid & (_lsr(ids, 16) == w) & (ids < _TAIL)
                rec = lax.shift_left(ids, 12) | pos
                plsc.store_compressed(binbuf.at[pl.ds(cur, 16)], rec, mask=m)
                return cur + plsc.all_reduce_population_count(m)[0]

            cursor = lax.fori_loop(0, 208, bin_grp, jnp.int32(0))

            def flush(t, _, w=w):
                pltpu.sync_copy(
                    binbuf.at[pl.ds(t * 128, 128)],
                    bucket.at[pl.ds((g * 16 + w) * _CAP + t * 128, 128)],
                )
                return 0

            lax.fori_loop(0, (cursor + 127) // 128, flush, 0)
            plsc.store_scatter(
                mycnt_v, [zeros16 + w], zeros16 + cursor, mask=iota < 1
            )

        pltpu.sync_copy(mycnt_v, counts.at[pl.ds(g * 128, 128)])

        def tail_bin(gg, cur):
            row = gg // 13
            grp = gg - row * 13
            ids, pos, valid = load_ids(row, grp)
            m = valid & (ids >= _TAIL)
            rec = lax.shift_left(ids, 12) | pos
            plsc.store_compressed(binbuf.at[pl.ds(cur, 16)], rec, mask=m)
            return cur + plsc.all_reduce_population_count(m)[0]

        cursor2 = lax.fori_loop(0, 208, tail_bin, jnp.int32(0))

        def tail_proc(t, _):
            v = binbuf[pl.ds(t, 16)]
            rec = v[0]
            trow = zeros16 + (_lsr(rec, 12) - _TAIL)
            rowbuf[0, pl.ds(0, 16)] = plsc.load_gather(tail_v, [trow, iota])
            rowbuf[0, pl.ds(16, 16)] = plsc.load_gather(
                tail_v, [trow, iota + 16]
            )
            pltpu.sync_copy(rowbuf.at[0], out1.at[tok_base + (rec & 4095)])
            return 0

        lax.fori_loop(0, cursor2, tail_proc, 0)

        plsc.subcore_barrier()

        pltpu.sync_copy(counts, cnt_v)
        for src in range(16):
            pltpu.async_copy(
                bucket.at[pl.ds(((c * 16 + src) * 16 + me) * _CAP, _SW)],
                work_v.at[src],
                sem,
            )
        for src in range(16):
            pltpu.make_async_copy(
                bucket.at[pl.ds(0, _SW)], work_v.at[0], sem
            ).wait()

        wbase = me * _WIN
        strips = (strip0, strip1)

        def sbase_load(sp):
            return jnp.minimum(wbase + sp * _SW, _CLAMP)

        pltpu.async_copy(tab_t.at[:, pl.ds(sbase_load(0), _SW)], strip0, sem)

        def subpair(pair, nscat0):
            nscat = nscat0
            for half in range(2):
                sp = pair * 2 + half
                cur = strips[half]
                nxt = strips[1 - half]

                @pl.when(sp + 1 < _NSP)
                def _(sp=sp, nxt=nxt):
                    pltpu.async_copy(
                        tab_t.at[:, pl.ds(sbase_load(sp + 1), _SW)], nxt, sem
                    )

                pltpu.make_async_copy(
                    tab_t.at[:, pl.ds(0, _SW)], cur, sem
                ).wait()
                sb = wbase + sp * _SW
                cb = sbase_load(sp)

                def src_loop(src, nscat1, cur=cur, sb=sb, cb=cb):
                    cidx = zeros16 + ((c * 16 + src) * 128 + me)
                    cnt = plsc.load_gather(cnt_v, [cidx])[0]

                    def chunk_loop(ck, hcur0, src=src, cnt=cnt, sb=sb, cb=cb):
                        @pl.when(ck > 0)
                        def _(src=src, ck=ck):
                            pltpu.sync_copy(
                                bucket.at[
                                    pl.ds(
                                        ((c * 16 + src) * 16 + me) * _CAP
                                        + ck * _SW,
                                        _SW,
                                    )
                                ],
                                work_v.at[src],
                            )

                        rem = jnp.minimum(cnt - ck * _SW, _SW)

                        def scan(tg, hcur, src=src, sb=sb, cb=cb, rem=rem):
                            recs = plsc.load_gather(
                                work_v, [zeros16 + src, tg * 16 + iota]
                            )
                            ids = _lsr(recs, 12)
                            m = (
                                ((tg * 16 + iota) < rem)
                                & (ids >= sb)
                                & ((ids - cb) < _SW)
                            )
                            plsc.store_compressed(
                                hitbuf.at[pl.ds(hcur, 16)], recs, mask=m
                            )
                            return hcur + plsc.all_reduce_population_count(m)[0]

                        return lax.fori_loop(0, (rem + 15) // 16, scan, hcur0)

                    nchunk = (cnt + _SW - 1) // _SW
                    hcur = lax.fori_loop(0, nchunk, chunk_loop, jnp.int32(0))

                    @pl.when(nchunk > 1)
                    def _(src=src):
                        pltpu.sync_copy(
                            bucket.at[
                                pl.ds(((c * 16 + src) * 16 + me) * _CAP, _SW)
                            ],
                            work_v.at[src],
                        )

                    def hgroup(hg, ns2, cur=cur, src=src, cb=cb, hcur=hcur):
                        @pl.when(ns2 >= 8)
                        def _():
                            pltpu.make_async_copy(
                                rowring.at[pl.ds(0, 16)],
                                out1.at[lax.iota(jnp.int32, 16)],
                                ssem,
                            ).wait()

                        slot = (ns2 % 8) * 16
                        recs16 = hitbuf[pl.ds(hg * 16, 16)]
                        pos16 = recs16 & 4095
                        cols16 = _lsr(recs16, 12) - cb
                        hv = (hg * 16 + iota) < hcur
                        dvec = jnp.where(
                            hv, (c * 16 + src) * _TPW + pos16, _N + me
                        )
                        hvi = hv.astype(jnp.int32)
                        for j in range(16):

                            @pl.when(hvi[j] == 1)
                            def _(j=j):
                                colv = zeros16 + cols16[j]
                                rowring[slot + j, pl.ds(0, 16)] = (
                                    plsc.load_gather(cur, [iota, colv])
                                )
                                rowring[slot + j, pl.ds(16, 16)] = (
                                    plsc.load_gather(cur, [iota + 16, colv])
                                )

                        pltpu.async_copy(
                            rowring.at[pl.ds(slot, 16)], out1.at[dvec], ssem
                        )
                        return ns2 + 1

                    return lax.fori_loop(0, (hcur + 15) // 16, hgroup, nscat1)

                nscat = lax.fori_loop(0, 16, src_loop, nscat)
            return nscat

        nscat_total = lax.fori_loop(0, _NSP // 2, subpair, jnp.int32(0))

        def drain(i, _):
            pltpu.make_async_copy(
                rowring.at[pl.ds(0, 16)],
                out1.at[lax.iota(jnp.int32, 16)],
                ssem,
            ).wait()
            return 0

        lax.fori_loop(0, jnp.minimum(nscat_total, 8), drain, 0)

    return gather_kernel


def kernel(doc_tokens, embedding):
    flat_idx = doc_tokens.reshape(_N).astype(jnp.int32)
    idxp = jnp.pad(flat_idx, (0, 384))
    rows, _, _ = _make_kernel()(embedding.T, idxp, embedding[_TAIL:])
    return rows[:_N, :_D].reshape(_B, _R, _K, _S, _D)


# speedup vs baseline: 1.7793x; 1.7793x over previous
import functools

import jax
import jax.numpy as jnp
from jax import lax
from jax.experimental import pallas as pl
from jax.experimental.pallas import tpu as pltpu
from jax.experimental.pallas import tpu_sc as plsc

_B, _R, _K, _S = 16, 4, 8, 200
_D = 32
_N = _B * _R * _K * _S
_TPW = 3200
_WIN = 65536
_SW = 2048
_NSP = _WIN // _SW
_TAIL = 999936
_CLAMP = _TAIL - _SW
_CAP = _TPW


def _lsr(x, n):
    return lax.shift_right_logical(x, n)


@functools.cache
def _make_kernel():
    info = plsc.get_sparse_core_info()
    nc, ns = info.num_cores, info.num_subcores
    assert nc == 2 and ns == 16
    nw = nc * ns
    mesh = plsc.VectorSubcoreMesh(core_axis_name="c", subcore_axis_name="s")

    @functools.partial(
        pl.kernel,
        mesh=mesh,
        compiler_params=pltpu.CompilerParams(needs_layout_passes=False),
        out_type=(
            jax.ShapeDtypeStruct((_N + 16, 128), jnp.float32),
            jax.ShapeDtypeStruct((nw * 16 * _CAP,), jnp.int32),
            jax.ShapeDtypeStruct((nw * 128,), jnp.int32),
        ),
        scratch_types=[
            pltpu.VMEM((16, 384), jnp.int32),
            pltpu.VMEM((3328,), jnp.int32),
            pltpu.VMEM((3328,), jnp.int32),
            pltpu.VMEM((16, 1024), jnp.int32),
            pltpu.VMEM((4096,), jnp.int32),
            pltpu.VMEM((128,), jnp.int32),
            pltpu.VMEM((_D, _SW), jnp.float32),
            pltpu.VMEM((16, 128), jnp.float32),
            pltpu.VMEM((64, 128), jnp.float32),
            pltpu.VMEM((64, _D), jnp.float32),
            pltpu.SemaphoreType.DMA,
            pltpu.SemaphoreType.DMA,
        ],
    )
    def gather_kernel(
        tab_t, idxp, tail_tab, out1, bucket, counts,
        idx_v, binbuf, hitbuf, work_v, cnt_v, mycnt_v,
        strip0, rowbuf, rowring, tail_v, sem, ssem,
    ):
        c = lax.axis_index("c")
        me = lax.axis_index("s")
        g = c * ns + me
        iota = lax.iota(jnp.int32, 16)
        zeros16 = jnp.zeros((16,), jnp.int32)

        tok_base = g * _TPW
        for row in range(16):
            align = (row * 200) // 128 * 128
            pltpu.sync_copy(
                idxp.at[pl.ds(tok_base + align, 384)], idx_v.at[row]
            )
        pltpu.sync_copy(tail_tab, tail_v)

        def load_ids(row, grp):
            shift = (row * 200) % 128
            ids = plsc.load_gather(
                idx_v, [zeros16 + row, shift + grp * 16 + iota]
            )
            pos = row * 200 + grp * 16 + iota
            valid = (grp * 16 + iota) < _S
            return ids, pos, valid

        for w in range(16):

            def bin_grp(gg, cur, w=w):
                row = gg // 13
                grp = gg - row * 13
                ids, pos, valid = load_ids(row, grp)
                m = valid & (_lsr(ids, 16) == w) & (ids < _TAIL)
                rec = lax.shift_left(ids, 12) | pos
                plsc.store_compressed(binbuf.at[pl.ds(cur, 16)], rec, mask=m)
                return cur + plsc.all_reduce_population_count(m)[0]

            cursor = lax.fori_loop(0, 208, bin_grp, jnp.int32(0))

            def flush(t, _, w=w):
                pltpu.sync_copy(
                    binbuf.at[pl.ds(t * 128, 128)],
                    bucket.at[pl.ds((g * 16 + w) * _CAP + t * 128, 128)],
                )
                return 0

            lax.fori_loop(0, (cursor + 127) // 128, flush, 0)
            plsc.store_scatter(
                mycnt_v, [zeros16 + w], zeros16 + cursor, mask=iota < 1
            )

        pltpu.sync_copy(mycnt_v, counts.at[pl.ds(g * 128, 128)])

        def tail_bin(gg, cur):
            row = gg // 13
            grp = gg - row * 13
            ids, pos, valid = load_ids(row, grp)
            m = valid & (ids >= _TAIL)
            rec = lax.shift_left(ids, 12) | pos
            plsc.store_compressed(binbuf.at[pl.ds(cur, 16)], rec, mask=m)
            return cur + plsc.all_reduce_population_count(m)[0]

        cursor2 = lax.fori_loop(0, 208, tail_bin, jnp.int32(0))

        def tail_proc(t, _):
            v = binbuf[pl.ds(t, 16)]
            rec = v[0]
            trow = zeros16 + (_lsr(rec, 12) - _TAIL)
            rowbuf[0, pl.ds(0, 16)] = plsc.load_gather(tail_v, [trow, iota])
            rowbuf[0, pl.ds(16, 16)] = plsc.load_gather(
                tail_v, [trow, iota + 16]
            )
            pltpu.sync_copy(rowbuf.at[0], out1.at[tok_base + (rec & 4095)])
            return 0

        lax.fori_loop(0, cursor2, tail_proc, 0)

        plsc.subcore_barrier()

        pltpu.sync_copy(counts, cnt_v)
        for src in range(16):
            pltpu.async_copy(
                bucket.at[pl.ds(((c * 16 + src) * 16 + me) * _CAP, 1024)],
                work_v.at[src],
                sem,
            )
        for src in range(16):
            pltpu.make_async_copy(
                bucket.at[pl.ds(0, 1024)], work_v.at[0], sem
            ).wait()

        wbase = me * _WIN

        def sbase_load(sp):
            return jnp.minimum(wbase + sp * _SW, _CLAMP)

        def subpair(sp, nscat0):
            nscat = nscat0
            if True:
                cur = strip0
                pltpu.sync_copy(
                    tab_t.at[:, pl.ds(sbase_load(sp), _SW)], cur
                )
                sb = wbase + sp * _SW
                cb = sbase_load(sp)

                def src_loop(src, nscat1, cur=cur, sb=sb, cb=cb):
                    cidx = zeros16 + ((c * 16 + src) * 128 + me)
                    cnt = plsc.load_gather(cnt_v, [cidx])[0]

                    def chunk_loop(ck, hcur0, src=src, cnt=cnt, sb=sb, cb=cb):
                        @pl.when(ck > 0)
                        def _(src=src, ck=ck):
                            pltpu.sync_copy(
                                bucket.at[
                                    pl.ds(
                                        ((c * 16 + src) * 16 + me) * _CAP
                                        + ck * 1024,
                                        1024,
                                    )
                                ],
                                work_v.at[src],
                            )

                        rem = jnp.minimum(cnt - ck * 1024, 1024)

                        def scan(tg, hcur, src=src, sb=sb, cb=cb, rem=rem):
                            recs = plsc.load_gather(
                                work_v, [zeros16 + src, tg * 16 + iota]
                            )
                            ids = _lsr(recs, 12)
                            m = (
                                ((tg * 16 + iota) < rem)
                                & (ids >= sb)
                                & ((ids - cb) < _SW)
                            )
                            plsc.store_compressed(
                                hitbuf.at[pl.ds(hcur, 16)], recs, mask=m
                            )
                            return hcur + plsc.all_reduce_population_count(m)[0]

                        return lax.fori_loop(0, (rem + 15) // 16, scan, hcur0)

                    nchunk = (cnt + 1023) // 1024
                    hcur = lax.fori_loop(0, nchunk, chunk_loop, jnp.int32(0))

                    @pl.when(nchunk > 1)
                    def _(src=src):
                        pltpu.sync_copy(
                            bucket.at[
                                pl.ds(((c * 16 + src) * 16 + me) * _CAP, 1024)
                            ],
                            work_v.at[src],
                        )

                    def hgroup(hg, ns2, cur=cur, src=src, cb=cb, hcur=hcur):
                        @pl.when(ns2 >= 4)
                        def _():
                            pltpu.make_async_copy(
                                rowring.at[pl.ds(0, 16)],
                                out1.at[lax.iota(jnp.int32, 16)],
                                ssem,
                            ).wait()

                        slot = (ns2 % 4) * 16
                        recs16 = hitbuf[pl.ds(hg * 16, 16)]
                        pos16 = recs16 & 4095
                        cols16 = _lsr(recs16, 12) - cb
                        hv = (hg * 16 + iota) < hcur
                        dvec = jnp.where(
                            hv, (c * 16 + src) * _TPW + pos16, _N + me
                        )
                        hvi = hv.astype(jnp.int32)
                        for j in range(16):

                            @pl.when(hvi[j] == 1)
                            def _(j=j):
                                colv = zeros16 + cols16[j]
                                rowring[slot + j, pl.ds(0, 16)] = (
                                    plsc.load_gather(cur, [iota, colv])
                                )
                                rowring[slot + j, pl.ds(16, 16)] = (
                                    plsc.load_gather(cur, [iota + 16, colv])
                                )

                        pltpu.async_copy(
                            rowring.at[pl.ds(slot, 16)], out1.at[dvec], ssem
                        )
                        return ns2 + 1

                    return lax.fori_loop(0, (hcur + 15) // 16, hgroup, nscat1)

                nscat = lax.fori_loop(0, 16, src_loop, nscat)
            return nscat

        nscat_total = lax.fori_loop(0, _NSP, subpair, jnp.int32(0))

        def drain(i, _):
            pltpu.make_async_copy(
                rowring.at[pl.ds(0, 16)],
                out1.at[lax.iota(jnp.int32, 16)],
                ssem,
            ).wait()
            return 0

        lax.fori_loop(0, jnp.minimum(nscat_total, 4), drain, 0)

    return gather_kernel


def kernel(doc_tokens, embedding):
    flat_idx = doc_tokens.reshape(_N).astype(jnp.int32)
    idxp = jnp.pad(flat_idx, (0, 384))
    rows, _, _ = _make_kernel()(embedding.T, idxp, embedding[_TAIL:])
    return rows[:_N, :_D].reshape(_B, _R, _K, _S, _D)


# speedup vs baseline: 2.6170x; 1.4708x over previous
import functools

import jax
import jax.numpy as jnp
from jax import lax
from jax.experimental import pallas as pl
from jax.experimental.pallas import tpu as pltpu
from jax.experimental.pallas import tpu_sc as plsc

_B, _R, _K, _S = 16, 4, 8, 200
_D = 32
_N = _B * _R * _K * _S


@functools.cache
def _make_gather(n_rows, d):
    info = plsc.get_sparse_core_info()
    nc, ns = info.num_cores, info.num_subcores
    nw = nc * ns
    assert n_rows % nw == 0
    per_w = n_rows // nw

    mesh = plsc.VectorSubcoreMesh(core_axis_name="c", subcore_axis_name="s")

    @functools.partial(
        pl.kernel,
        mesh=mesh,
        compiler_params=pltpu.CompilerParams(use_tc_tiling_on_sc=False),
        out_type=jax.ShapeDtypeStruct((n_rows, d), jnp.float32),
        scratch_types=[
            pltpu.VMEM((per_w,), jnp.int32),
            pltpu.VMEM((per_w, d), jnp.float32),
            pltpu.SemaphoreType.DMA,
        ],
    )
    def gather_kernel(table_hbm, idx_hbm, out_hbm, idx_v, rows_v, sem):
        wid = lax.axis_index("s") * nc + lax.axis_index("c")
        base = wid * per_w
        pltpu.sync_copy(idx_hbm.at[pl.ds(base, per_w)], idx_v)
        pltpu.async_copy(table_hbm.at[idx_v], rows_v, sem).wait()
        pltpu.sync_copy(rows_v, out_hbm.at[pl.ds(base, per_w)])

    return gather_kernel


def kernel(doc_tokens, embedding):
    flat_idx = doc_tokens.reshape(_N).astype(jnp.int32)
    flat_table = embedding
    rows = _make_gather(_N, _D)(flat_table, flat_idx)
    return rows.reshape(_B, _R, _K, _S, _D)
